# baseline (speedup 1.0000x reference)
import jax
import jax.numpy as jnp
from jax import lax
from jax.experimental import pallas as pl
from jax.experimental.pallas import tpu as pltpu
from jax.experimental.pallas import tpu_sc as plsc

S = 200
B = 4096
D = 64
L = 16
V = 1000000

NC = 2
NS = 16
NW = NC * NS
BW = B // NW
NG = BW // L
N_ROWS2 = V // 2
NFULL = V // 128


def _tbody(tt_hbm, out_hbm, vin0, vin1, vout0, vout1, vin2, vout2,
           g0, g1, w0, w1):
    wid = lax.axis_index("s") * NC + lax.axis_index("c")

    iota = lax.iota(jnp.int32, L)
    hvec = jax.lax.shift_left(iota & 1, 6)
    ivecs = tuple(g * 8 + jax.lax.shift_right_logical(iota, 1)
                  for g in range(NG))
    tvecs = tuple(g * L + iota for g in range(NG))

    def chunk_of(k):
        return wid + NW * k

    def prep(k, vin_v, gsem):
        c = chunk_of(k)
        pltpu.async_copy(tt_hbm.at[:, pl.ds(c * 128, 128)], vin_v, gsem)

    def wait_g(sem, vin_v):
        pltpu.make_async_copy(tt_hbm.at[:, pl.ds(0, 128)], vin_v, sem).wait()

    def wait_w(sem, vout_v):
        pltpu.make_async_copy(vout_v, out_hbm.at[pl.ds(0, 64)], sem).wait()

    def transpose(vin_v, vout_v, ng):
        def r_body(r, carry):
            rot = (iota + r) & 15
            for dg in range(4):
                drot = dg * L + rot
                cvec = hvec + drot
                for g in range(ng):
                    vals = plsc.load_gather(vin_v, [drot, tvecs[g]])
                    plsc.store_scatter(vout_v, [ivecs[g], cvec], vals)
            return carry

        lax.fori_loop(0, L, r_body, 0)

    nmine = jax.lax.div(NFULL - wid + NW - 1, NW)

    prep(0, vin0, g0)
    prep(1, vin1, g1)
    bufs = ((vin0, vout0, g0, w0), (vin1, vout1, g1, w1))

    def step(i, carry):
        for bsel in range(2):
            k = 2 * i + bsel
            vin_v, vout_v, gsem, wsem = bufs[bsel]

            @pl.when(k < nmine)
            def _():
                @pl.when(k >= 2)
                def _():
                    wait_w(wsem, vout_v)

                wait_g(gsem, vin_v)
                transpose(vin_v, vout_v, NG)

                @pl.when(k + 2 < nmine)
                def _():
                    prep(k + 2, vin_v, gsem)

                c = chunk_of(k)
                pltpu.async_copy(vout_v, out_hbm.at[pl.ds(c * 64, 64)], wsem)
        return carry

    lax.fori_loop(0, 123, step, 0)

    @pl.when(nmine >= 1)
    def _():
        wait_w(w0, vout0)

    @pl.when(nmine >= 2)
    def _():
        wait_w(w1, vout1)

    @pl.when(wid == NW - 1)
    def _():
        pltpu.sync_copy(tt_hbm.at[:, pl.ds(NFULL * 128, 64)], vin2)
        transpose(vin2, vout2, NG // 2)
        pltpu.sync_copy(vout2, out_hbm.at[pl.ds(NFULL * 64, 32)])


def _body(src_hbm, table_hbm, pe_hbm, out_hbm,
          idx_v, pe_v, slab0, slab1, tout0, tout1, g0, g1, w0, w1):
    wid = lax.axis_index("s") * NC + lax.axis_index("c")
    bcol = wid * BW

    pltpu.sync_copy(src_hbm.at[:, pl.ds(bcol, BW)], idx_v)
    pltpu.sync_copy(pe_hbm, pe_v)

    iota = lax.iota(jnp.int32, L)
    bconst = tuple(g * L + iota for g in range(NG))

    def prep_gather(s, slab_v, gsem):
        pltpu.async_copy(table_hbm.at[idx_v.at[s]], slab_v, gsem)

    def compute(s, slab_v, tout_v):
        pe16 = tuple(pe_v[s, pl.ds(dg * L, L)] for dg in range(4))

        def r_body(r, carry):
            rot = (iota + r) & 15
            for dg in range(4):
                drot = dg * L + rot
                perot = pe16[dg].at[rot].get(mode="promise_in_bounds")
                for g in range(NG):
                    vals = plsc.load_gather(slab_v, [bconst[g], drot]) + perot
                    plsc.store_scatter(tout_v, [drot, bconst[g]], vals)
            return carry

        lax.fori_loop(0, L, r_body, 0)

    def emit(s, tout_v, wsem):
        pltpu.async_copy(tout_v, out_hbm.at[s, :, pl.ds(bcol, BW)], wsem)

    def wait_g(sem, slab_v):
        pltpu.make_async_copy(table_hbm.at[idx_v.at[0]], slab_v, sem).wait()

    def wait_w(sem, tout_v):
        pltpu.make_async_copy(tout_v, out_hbm.at[0, :, pl.ds(bcol, BW)],
                              sem).wait()

    prep_gather(0, slab0, g0)
    prep_gather(1, slab1, g1)

    bufs = ((slab0, tout0, g0, w0), (slab1, tout1, g1, w1))

    def step(i, carry):
        for bsel in range(2):
            s = 2 * i + bsel
            slab_v, tout_v, gsem, wsem = bufs[bsel]

            @pl.when(s >= 2)
            def _():
                wait_w(wsem, tout_v)

            wait_g(gsem, slab_v)
            compute(s, slab_v, tout_v)

            @pl.when(s + 2 < S)
            def _():
                prep_gather(s + 2, slab_v, gsem)

            emit(s, tout_v, wsem)
        return carry

    lax.fori_loop(0, S // 2, step, 0)

    wait_w(w0, tout0)
    wait_w(w1, tout1)


@jax.jit
def _pe_embed(src, table_t, pe2d):
    mesh = plsc.VectorSubcoreMesh(core_axis_name="c", subcore_axis_name="s")

    tkern = pl.kernel(
        _tbody,
        out_type=jax.ShapeDtypeStruct((N_ROWS2, 2 * D), jnp.float32),
        mesh=mesh,
        scratch_types=[
            pltpu.VMEM((D, 2 * D), jnp.float32),
            pltpu.VMEM((D, 2 * D), jnp.float32),
            pltpu.VMEM((D, 2 * D), jnp.float32),
            pltpu.VMEM((D, 2 * D), jnp.float32),
            pltpu.VMEM((D, D), jnp.float32),
            pltpu.VMEM((D // 2, 2 * D), jnp.float32),
            pltpu.SemaphoreType.DMA,
            pltpu.SemaphoreType.DMA,
            pltpu.SemaphoreType.DMA,
            pltpu.SemaphoreType.DMA,
        ],
        compiler_params=pltpu.CompilerParams(
            use_tc_tiling_on_sc=True, needs_layout_passes=False
        ),
    )

    mainkern = pl.kernel(
        _body,
        out_type=jax.ShapeDtypeStruct((S, D, B), jnp.float32),
        mesh=mesh,
        scratch_types=[
            pltpu.VMEM((S, BW), jnp.int32),
            pltpu.VMEM((S, D), jnp.float32),
            pltpu.VMEM((BW, D), jnp.float32),
            pltpu.VMEM((BW, D), jnp.float32),
            pltpu.VMEM((D, BW), jnp.float32),
            pltpu.VMEM((D, BW), jnp.float32),
            pltpu.SemaphoreType.DMA,
            pltpu.SemaphoreType.DMA,
            pltpu.SemaphoreType.DMA,
            pltpu.SemaphoreType.DMA,
        ],
        compiler_params=pltpu.CompilerParams(
            use_tc_tiling_on_sc=False, needs_layout_passes=False
        ),
    )

    t2 = tkern(table_t)
    t3 = t2.reshape(V, D)
    return mainkern(src, t3, pe2d)


def kernel(src, table, pe):
    src = src.astype(jnp.int32)
    table_t = table.T
    pe2d = pe.reshape(S, D)
    out_t = _pe_embed(src, table_t, pe2d)
    return out_t.transpose(0, 2, 1)


# speedup vs baseline: 1.0631x; 1.0631x over previous
import jax
import jax.numpy as jnp
from jax import lax
from jax.experimental import pallas as pl
from jax.experimental.pallas import tpu as pltpu
from jax.experimental.pallas import tpu_sc as plsc

S = 200
B = 4096
D = 64
L = 16
V = 1000000

NC = 2
NS = 16
NW = NC * NS
BW = B // NW
NG = BW // L
N_ROWS2 = V // 2
NFULL = V // 128


def _tbody(tt_hbm, out_hbm, vin0, vin1, vout0, vout1, vin2, vout2,
           g0, g1, w0, w1):
    wid = lax.axis_index("s") * NC + lax.axis_index("c")

    iota = lax.iota(jnp.int32, L)
    hvec = jax.lax.shift_left(iota & 1, 6)
    ivecs = tuple(g * 8 + jax.lax.shift_right_logical(iota, 1)
                  for g in range(NG))
    tvecs = tuple(g * L + iota for g in range(NG))

    def chunk_of(k):
        return wid + NW * k

    def prep(k, vin_v, gsem):
        c = chunk_of(k)
        pltpu.async_copy(tt_hbm.at[:, pl.ds(c * 128, 128)], vin_v, gsem)

    def wait_g(sem, vin_v):
        pltpu.make_async_copy(tt_hbm.at[:, pl.ds(0, 128)], vin_v, sem).wait()

    def wait_w(sem, vout_v):
        pltpu.make_async_copy(vout_v, out_hbm.at[pl.ds(0, 64)], sem).wait()

    def transpose(vin_v, vout_v, ng):
        def r_body(r4, carry):
            for ru in range(4):
                rot = (iota + (r4 * 4 + ru)) & 15
                for dg in range(4):
                    drot = dg * L + rot
                    cvec = hvec + drot
                    for g in range(ng):
                        vals = plsc.load_gather(vin_v, [drot, tvecs[g]])
                        plsc.store_scatter(vout_v, [ivecs[g], cvec], vals)
            return carry

        lax.fori_loop(0, 4, r_body, 0)

    nmine = jax.lax.div(NFULL - wid + NW - 1, NW)

    prep(0, vin0, g0)
    prep(1, vin1, g1)
    bufs = ((vin0, vout0, g0, w0), (vin1, vout1, g1, w1))

    def step(i, carry):
        for bsel in range(2):
            k = 2 * i + bsel
            vin_v, vout_v, gsem, wsem = bufs[bsel]

            @pl.when(k < nmine)
            def _():
                @pl.when(k >= 2)
                def _():
                    wait_w(wsem, vout_v)

                wait_g(gsem, vin_v)
                transpose(vin_v, vout_v, NG)

                @pl.when(k + 2 < nmine)
                def _():
                    prep(k + 2, vin_v, gsem)

                c = chunk_of(k)
                pltpu.async_copy(vout_v, out_hbm.at[pl.ds(c * 64, 64)], wsem)
        return carry

    lax.fori_loop(0, 123, step, 0)

    @pl.when(nmine >= 1)
    def _():
        wait_w(w0, vout0)

    @pl.when(nmine >= 2)
    def _():
        wait_w(w1, vout1)

    @pl.when(wid == NW - 1)
    def _():
        pltpu.sync_copy(tt_hbm.at[:, pl.ds(NFULL * 128, 64)], vin2)
        transpose(vin2, vout2, NG // 2)
        pltpu.sync_copy(vout2, out_hbm.at[pl.ds(NFULL * 64, 32)])


def _body(src_hbm, table_hbm, pe_hbm, out_hbm,
          idx_v, pe_v, slab0, slab1, tout0, tout1, g0, g1, w0, w1):
    wid = lax.axis_index("s") * NC + lax.axis_index("c")
    bcol = wid * BW

    pltpu.sync_copy(src_hbm.at[:, pl.ds(bcol, BW)], idx_v)
    pltpu.sync_copy(pe_hbm, pe_v)

    iota = lax.iota(jnp.int32, L)
    bconst = tuple(g * L + iota for g in range(NG))

    def prep_gather(s, slab_v, gsem):
        pltpu.async_copy(table_hbm.at[idx_v.at[s]], slab_v, gsem)

    def compute(s, slab_v, tout_v):
        pe16 = tuple(pe_v[s, pl.ds(dg * L, L)] for dg in range(4))

        def r_body(r4, carry):
            for ru in range(4):
                rot = (iota + (r4 * 4 + ru)) & 15
                for dg in range(4):
                    drot = dg * L + rot
                    perot = pe16[dg].at[rot].get(mode="promise_in_bounds")
                    for g in range(NG):
                        vals = plsc.load_gather(
                            slab_v, [bconst[g], drot]) + perot
                        plsc.store_scatter(tout_v, [drot, bconst[g]], vals)
            return carry

        lax.fori_loop(0, 4, r_body, 0)

    def emit(s, tout_v, wsem):
        pltpu.async_copy(tout_v, out_hbm.at[s, :, pl.ds(bcol, BW)], wsem)

    def wait_g(sem, slab_v):
        pltpu.make_async_copy(table_hbm.at[idx_v.at[0]], slab_v, sem).wait()

    def wait_w(sem, tout_v):
        pltpu.make_async_copy(tout_v, out_hbm.at[0, :, pl.ds(bcol, BW)],
                              sem).wait()

    prep_gather(0, slab0, g0)
    prep_gather(1, slab1, g1)

    bufs = ((slab0, tout0, g0, w0), (slab1, tout1, g1, w1))

    def step(i, carry):
        for bsel in range(2):
            s = 2 * i + bsel
            slab_v, tout_v, gsem, wsem = bufs[bsel]

            @pl.when(s >= 2)
            def _():
                wait_w(wsem, tout_v)

            wait_g(gsem, slab_v)
            compute(s, slab_v, tout_v)

            @pl.when(s + 2 < S)
            def _():
                prep_gather(s + 2, slab_v, gsem)

            emit(s, tout_v, wsem)
        return carry

    lax.fori_loop(0, S // 2, step, 0)

    wait_w(w0, tout0)
    wait_w(w1, tout1)


@jax.jit
def _pe_embed(src, table_t, pe2d):
    mesh = plsc.VectorSubcoreMesh(core_axis_name="c", subcore_axis_name="s")

    tkern = pl.kernel(
        _tbody,
        out_type=jax.ShapeDtypeStruct((N_ROWS2, 2 * D), jnp.float32),
        mesh=mesh,
        scratch_types=[
            pltpu.VMEM((D, 2 * D), jnp.float32),
            pltpu.VMEM((D, 2 * D), jnp.float32),
            pltpu.VMEM((D, 2 * D), jnp.float32),
            pltpu.VMEM((D, 2 * D), jnp.float32),
            pltpu.VMEM((D, D), jnp.float32),
            pltpu.VMEM((D // 2, 2 * D), jnp.float32),
            pltpu.SemaphoreType.DMA,
            pltpu.SemaphoreType.DMA,
            pltpu.SemaphoreType.DMA,
            pltpu.SemaphoreType.DMA,
        ],
        compiler_params=pltpu.CompilerParams(
            use_tc_tiling_on_sc=True, needs_layout_passes=False
        ),
    )

    mainkern = pl.kernel(
        _body,
        out_type=jax.ShapeDtypeStruct((S, D, B), jnp.float32),
        mesh=mesh,
        scratch_types=[
            pltpu.VMEM((S, BW), jnp.int32),
            pltpu.VMEM((S, D), jnp.float32),
            pltpu.VMEM((BW, D), jnp.float32),
            pltpu.VMEM((BW, D), jnp.float32),
            pltpu.VMEM((D, BW), jnp.float32),
            pltpu.VMEM((D, BW), jnp.float32),
            pltpu.SemaphoreType.DMA,
            pltpu.SemaphoreType.DMA,
            pltpu.SemaphoreType.DMA,
            pltpu.SemaphoreType.DMA,
        ],
        compiler_params=pltpu.CompilerParams(
            use_tc_tiling_on_sc=False, needs_layout_passes=False
        ),
    )

    t2 = tkern(table_t)
    t3 = t2.reshape(V, D)
    return mainkern(src, t3, pe2d)


def kernel(src, table, pe):
    src = src.astype(jnp.int32)
    table_t = table.T
    pe2d = pe.reshape(S, D)
    out_t = _pe_embed(src, table_t, pe2d)
    return out_t.transpose(0, 2, 1)


# speedup vs baseline: 1.7084x; 1.6070x over previous
import jax
import jax.numpy as jnp
from jax import lax
from jax.experimental import pallas as pl
from jax.experimental.pallas import tpu as pltpu
from jax.experimental.pallas import tpu_sc as plsc

S = 200
B = 4096
D = 64
L = 16
V = 1000000

NC = 2
NS = 16
NW = NC * NS
BW = B // NW
NG = BW // L
N_ROWS2 = V // 2
NFULL = V // 128


def _tbody(tt_hbm, out_hbm, vin0, vin1, vout0, vout1, vin2, vout2,
           g0, g1, w0, w1):
    wid = lax.axis_index("s") * NC + lax.axis_index("c")

    iota = lax.iota(jnp.int32, L)
    hvec = jax.lax.shift_left(iota & 1, 6)
    ivecs = tuple(g * 8 + jax.lax.shift_right_logical(iota, 1)
                  for g in range(NG))
    tvecs = tuple(g * L + iota for g in range(NG))

    def chunk_of(k):
        return wid + NW * k

    def prep(k, vin_v, gsem):
        c = chunk_of(k)
        pltpu.async_copy(tt_hbm.at[:, pl.ds(c * 128, 128)], vin_v, gsem)

    def wait_g(sem, vin_v):
        pltpu.make_async_copy(tt_hbm.at[:, pl.ds(0, 128)], vin_v, sem).wait()

    def wait_w(sem, vout_v):
        pltpu.make_async_copy(vout_v, out_hbm.at[pl.ds(0, 64)], sem).wait()

    def transpose(vin_v, vout_v, ng):
        @plsc.parallel_loop(0, L, unroll=4)
        def _(r):
            rot = (iota + r) & 15
            for dg in range(4):
                drot = dg * L + rot
                cvec = hvec + drot
                for g in range(ng):
                    vals = plsc.load_gather(vin_v, [drot, tvecs[g]])
                    plsc.store_scatter(vout_v, [ivecs[g], cvec], vals)

    nmine = jax.lax.div(NFULL - wid + NW - 1, NW)

    prep(0, vin0, g0)
    prep(1, vin1, g1)
    bufs = ((vin0, vout0, g0, w0), (vin1, vout1, g1, w1))

    def step(i, carry):
        for bsel in range(2):
            k = 2 * i + bsel
            vin_v, vout_v, gsem, wsem = bufs[bsel]

            @pl.when(k < nmine)
            def _():
                @pl.when(k >= 2)
                def _():
                    wait_w(wsem, vout_v)

                wait_g(gsem, vin_v)
                transpose(vin_v, vout_v, NG)

                @pl.when(k + 2 < nmine)
                def _():
                    prep(k + 2, vin_v, gsem)

                c = chunk_of(k)
                pltpu.async_copy(vout_v, out_hbm.at[pl.ds(c * 64, 64)], wsem)
        return carry

    lax.fori_loop(0, 123, step, 0)

    @pl.when(nmine >= 1)
    def _():
        wait_w(w0, vout0)

    @pl.when(nmine >= 2)
    def _():
        wait_w(w1, vout1)

    @pl.when(wid == NW - 1)
    def _():
        pltpu.sync_copy(tt_hbm.at[:, pl.ds(NFULL * 128, 64)], vin2)
        transpose(vin2, vout2, NG // 2)
        pltpu.sync_copy(vout2, out_hbm.at[pl.ds(NFULL * 64, 32)])


def _body(src_hbm, table_hbm, pe_hbm, out_hbm,
          idx_v, pe_v, slab0, slab1, tout0, tout1, g0, g1, w0, w1):
    wid = lax.axis_index("s") * NC + lax.axis_index("c")
    bcol = wid * BW

    pltpu.sync_copy(src_hbm.at[:, pl.ds(bcol, BW)], idx_v)
    pltpu.sync_copy(pe_hbm, pe_v)

    iota = lax.iota(jnp.int32, L)
    bconst = tuple(g * L + iota for g in range(NG))

    def prep_gather(s, slab_v, gsem):
        pltpu.async_copy(table_hbm.at[idx_v.at[s]], slab_v, gsem)

    def compute(s, slab_v, tout_v):
        pe16 = tuple(pe_v[s, pl.ds(dg * L, L)] for dg in range(4))

        @plsc.parallel_loop(0, L, unroll=4)
        def _(r):
            rot = (iota + r) & 15
            for dg in range(4):
                drot = dg * L + rot
                perot = pe16[dg].at[rot].get(mode="promise_in_bounds")
                for g in range(NG):
                    vals = plsc.load_gather(slab_v, [bconst[g], drot]) + perot
                    plsc.store_scatter(tout_v, [drot, bconst[g]], vals)

    def emit(s, tout_v, wsem):
        pltpu.async_copy(tout_v, out_hbm.at[s, :, pl.ds(bcol, BW)], wsem)

    def wait_g(sem, slab_v):
        pltpu.make_async_copy(table_hbm.at[idx_v.at[0]], slab_v, sem).wait()

    def wait_w(sem, tout_v):
        pltpu.make_async_copy(tout_v, out_hbm.at[0, :, pl.ds(bcol, BW)],
                              sem).wait()

    prep_gather(0, slab0, g0)
    prep_gather(1, slab1, g1)

    bufs = ((slab0, tout0, g0, w0), (slab1, tout1, g1, w1))

    def step(i, carry):
        for bsel in range(2):
            s = 2 * i + bsel
            slab_v, tout_v, gsem, wsem = bufs[bsel]

            @pl.when(s >= 2)
            def _():
                wait_w(wsem, tout_v)

            wait_g(gsem, slab_v)
            compute(s, slab_v, tout_v)

            @pl.when(s + 2 < S)
            def _():
                prep_gather(s + 2, slab_v, gsem)

            emit(s, tout_v, wsem)
        return carry

    lax.fori_loop(0, S // 2, step, 0)

    wait_w(w0, tout0)
    wait_w(w1, tout1)


@jax.jit
def _pe_embed(src, table_t, pe2d):
    mesh = plsc.VectorSubcoreMesh(core_axis_name="c", subcore_axis_name="s")

    tkern = pl.kernel(
        _tbody,
        out_type=jax.ShapeDtypeStruct((N_ROWS2, 2 * D), jnp.float32),
        mesh=mesh,
        scratch_types=[
            pltpu.VMEM((D, 2 * D), jnp.float32),
            pltpu.VMEM((D, 2 * D), jnp.float32),
            pltpu.VMEM((D, 2 * D), jnp.float32),
            pltpu.VMEM((D, 2 * D), jnp.float32),
            pltpu.VMEM((D, D), jnp.float32),
            pltpu.VMEM((D // 2, 2 * D), jnp.float32),
            pltpu.SemaphoreType.DMA,
            pltpu.SemaphoreType.DMA,
            pltpu.SemaphoreType.DMA,
            pltpu.SemaphoreType.DMA,
        ],
        compiler_params=pltpu.CompilerParams(
            use_tc_tiling_on_sc=True, needs_layout_passes=False
        ),
    )

    mainkern = pl.kernel(
        _body,
        out_type=jax.ShapeDtypeStruct((S, D, B), jnp.float32),
        mesh=mesh,
        scratch_types=[
            pltpu.VMEM((S, BW), jnp.int32),
            pltpu.VMEM((S, D), jnp.float32),
            pltpu.VMEM((BW, D), jnp.float32),
            pltpu.VMEM((BW, D), jnp.float32),
            pltpu.VMEM((D, BW), jnp.float32),
            pltpu.VMEM((D, BW), jnp.float32),
            pltpu.SemaphoreType.DMA,
            pltpu.SemaphoreType.DMA,
            pltpu.SemaphoreType.DMA,
            pltpu.SemaphoreType.DMA,
        ],
        compiler_params=pltpu.CompilerParams(
            use_tc_tiling_on_sc=False, needs_layout_passes=False
        ),
    )

    t2 = tkern(table_t)
    t3 = t2.reshape(V, D)
    return mainkern(src, t3, pe2d)


def kernel(src, table, pe):
    src = src.astype(jnp.int32)
    table_t = table.T
    pe2d = pe.reshape(S, D)
    out_t = _pe_embed(src, table_t, pe2d)
    return out_t.transpose(0, 2, 1)


# speedup vs baseline: 1.8835x; 1.1025x over previous
import jax
import jax.numpy as jnp
from jax import lax
from jax.experimental import pallas as pl
from jax.experimental.pallas import tpu as pltpu
from jax.experimental.pallas import tpu_sc as plsc

S = 200
B = 4096
D = 64
L = 16
V = 1000000

NC = 2
NS = 16
NW = NC * NS
BW = B // NW
NG = BW // L
N_ROWS2 = V // 2
NFULL = V // 128


def _tbody(tt_hbm, out_hbm, vin0, vin1, vout0, vout1, vin2, vout2,
           g0, g1, w0, w1):
    wid = lax.axis_index("s") * NC + lax.axis_index("c")

    iota = lax.iota(jnp.int32, L)
    hvec = jax.lax.shift_left(iota & 1, 6)
    ivecs = tuple(g * 8 + jax.lax.shift_right_logical(iota, 1)
                  for g in range(NG))
    tvecs = tuple(g * L + iota for g in range(NG))

    def chunk_of(k):
        return wid + NW * k

    def prep(k, vin_v, gsem):
        c = chunk_of(k)
        pltpu.async_copy(tt_hbm.at[:, pl.ds(c * 128, 128)], vin_v, gsem)

    def wait_g(sem, vin_v):
        pltpu.make_async_copy(tt_hbm.at[:, pl.ds(0, 128)], vin_v, sem).wait()

    def wait_w(sem, vout_v):
        pltpu.make_async_copy(vout_v, out_hbm.at[pl.ds(0, 64)], sem).wait()

    def transpose(vin_v, vout_v, ng):
        @plsc.parallel_loop(0, L, unroll=8)
        def _(r):
            rot = (iota + r) & 15
            for dg in range(4):
                drot = dg * L + rot
                cvec = hvec + drot
                for g in range(ng):
                    vals = plsc.load_gather(vin_v, [drot, tvecs[g]])
                    plsc.store_scatter(vout_v, [ivecs[g], cvec], vals)

    nmine = jax.lax.div(NFULL - wid + NW - 1, NW)

    prep(0, vin0, g0)
    prep(1, vin1, g1)
    bufs = ((vin0, vout0, g0, w0), (vin1, vout1, g1, w1))

    def step(i, carry):
        for bsel in range(2):
            k = 2 * i + bsel
            vin_v, vout_v, gsem, wsem = bufs[bsel]

            @pl.when(k < nmine)
            def _():
                @pl.when(k >= 2)
                def _():
                    wait_w(wsem, vout_v)

                wait_g(gsem, vin_v)
                transpose(vin_v, vout_v, NG)

                @pl.when(k + 2 < nmine)
                def _():
                    prep(k + 2, vin_v, gsem)

                c = chunk_of(k)
                pltpu.async_copy(vout_v, out_hbm.at[pl.ds(c * 64, 64)], wsem)
        return carry

    lax.fori_loop(0, 123, step, 0)

    @pl.when(nmine >= 1)
    def _():
        wait_w(w0, vout0)

    @pl.when(nmine >= 2)
    def _():
        wait_w(w1, vout1)

    @pl.when(wid == NW - 1)
    def _():
        pltpu.sync_copy(tt_hbm.at[:, pl.ds(NFULL * 128, 64)], vin2)
        transpose(vin2, vout2, NG // 2)
        pltpu.sync_copy(vout2, out_hbm.at[pl.ds(NFULL * 64, 32)])


def _body(src_hbm, table_hbm, pe_hbm, out_hbm,
          idx_v, pe_v, slab0, slab1, tout0, tout1, g0, g1, w0, w1):
    wid = lax.axis_index("s") * NC + lax.axis_index("c")
    bcol = wid * BW

    pltpu.sync_copy(src_hbm.at[:, pl.ds(bcol, BW)], idx_v)
    pltpu.sync_copy(pe_hbm, pe_v)

    iota = lax.iota(jnp.int32, L)
    bconst = tuple(g * L + iota for g in range(NG))

    def prep_gather(s, slab_v, gsem):
        pltpu.async_copy(table_hbm.at[idx_v.at[s]], slab_v, gsem)

    def compute(s, slab_v, tout_v):
        pe16 = tuple(pe_v[s, pl.ds(dg * L, L)] for dg in range(4))

        @plsc.parallel_loop(0, L, unroll=8)
        def _(r):
            rot = (iota + r) & 15
            for dg in range(4):
                drot = dg * L + rot
                perot = pe16[dg].at[rot].get(mode="promise_in_bounds")
                for g in range(NG):
                    vals = plsc.load_gather(slab_v, [bconst[g], drot]) + perot
                    plsc.store_scatter(tout_v, [drot, bconst[g]], vals)

    def emit(s, tout_v, wsem):
        pltpu.async_copy(tout_v, out_hbm.at[s, :, pl.ds(bcol, BW)], wsem)

    def wait_g(sem, slab_v):
        pltpu.make_async_copy(table_hbm.at[idx_v.at[0]], slab_v, sem).wait()

    def wait_w(sem, tout_v):
        pltpu.make_async_copy(tout_v, out_hbm.at[0, :, pl.ds(bcol, BW)],
                              sem).wait()

    prep_gather(0, slab0, g0)
    prep_gather(1, slab1, g1)

    bufs = ((slab0, tout0, g0, w0), (slab1, tout1, g1, w1))

    def step(i, carry):
        for bsel in range(2):
            s = 2 * i + bsel
            slab_v, tout_v, gsem, wsem = bufs[bsel]

            @pl.when(s >= 2)
            def _():
                wait_w(wsem, tout_v)

            wait_g(gsem, slab_v)
            compute(s, slab_v, tout_v)

            @pl.when(s + 2 < S)
            def _():
                prep_gather(s + 2, slab_v, gsem)

            emit(s, tout_v, wsem)
        return carry

    lax.fori_loop(0, S // 2, step, 0)

    wait_w(w0, tout0)
    wait_w(w1, tout1)


@jax.jit
def _pe_embed(src, table_t, pe2d):
    mesh = plsc.VectorSubcoreMesh(core_axis_name="c", subcore_axis_name="s")

    tkern = pl.kernel(
        _tbody,
        out_type=jax.ShapeDtypeStruct((N_ROWS2, 2 * D), jnp.float32),
        mesh=mesh,
        scratch_types=[
            pltpu.VMEM((D, 2 * D), jnp.float32),
            pltpu.VMEM((D, 2 * D), jnp.float32),
            pltpu.VMEM((D, 2 * D), jnp.float32),
            pltpu.VMEM((D, 2 * D), jnp.float32),
            pltpu.VMEM((D, D), jnp.float32),
            pltpu.VMEM((D // 2, 2 * D), jnp.float32),
            pltpu.SemaphoreType.DMA,
            pltpu.SemaphoreType.DMA,
            pltpu.SemaphoreType.DMA,
            pltpu.SemaphoreType.DMA,
        ],
        compiler_params=pltpu.CompilerParams(
            use_tc_tiling_on_sc=True, needs_layout_passes=False
        ),
    )

    mainkern = pl.kernel(
        _body,
        out_type=jax.ShapeDtypeStruct((S, D, B), jnp.float32),
        mesh=mesh,
        scratch_types=[
            pltpu.VMEM((S, BW), jnp.int32),
            pltpu.VMEM((S, D), jnp.float32),
            pltpu.VMEM((BW, D), jnp.float32),
            pltpu.VMEM((BW, D), jnp.float32),
            pltpu.VMEM((D, BW), jnp.float32),
            pltpu.VMEM((D, BW), jnp.float32),
            pltpu.SemaphoreType.DMA,
            pltpu.SemaphoreType.DMA,
            pltpu.SemaphoreType.DMA,
            pltpu.SemaphoreType.DMA,
        ],
        compiler_params=pltpu.CompilerParams(
            use_tc_tiling_on_sc=False, needs_layout_passes=False
        ),
    )

    t2 = tkern(table_t)
    t3 = t2.reshape(V, D)
    return mainkern(src, t3, pe2d)


def kernel(src, table, pe):
    src = src.astype(jnp.int32)
    table_t = table.T
    pe2d = pe.reshape(S, D)
    out_t = _pe_embed(src, table_t, pe2d)
    return out_t.transpose(0, 2, 1)
